# baseline (speedup 1.0000x reference)
import jax
import jax.numpy as jnp
from jax import lax
from jax.experimental import pallas as pl
from jax.experimental.pallas import tpu as pltpu
from jax.experimental.pallas import tpu_sc as plsc

KNN = 16
_SC_CORES = 2
_SC_SUBCORES = 16
_NW = _SC_CORES * _SC_SUBCORES
_NEG_HUGE = -3.4e38


def _leaky(x):
    return jnp.where(x >= 0, x, 0.2 * x)


def _mm(a, w):
    return lax.dot_general(
        a, w, (((1,), (1,)), ((), ())),
        preferred_element_type=jnp.float32,
        precision=lax.Precision.DEFAULT,
    )


_DIST_R = 256


def _dist_body(xtf_ref, xtr_ref, wb_ref, b_ref, d_ref, yb_ref):
    xtf = xtf_ref[0]
    xtr = xtr_ref[0]
    inner = -2.0 * lax.dot_general(
        xtr, xtf, (((1,), (1,)), ((), ())),
        preferred_element_type=jnp.float32, precision=lax.Precision.DEFAULT)
    xxf = jnp.sum(xtf * xtf, axis=1)
    xxr = jnp.sum(xtr * xtr, axis=1)
    d_ref[0] = -xxr[:, None] - inner - xxf[None, :]
    yb_ref[0] = _mm(xtr, wb_ref[...]) + b_ref[...][None, :]


def _dist_yb(xt, wb, b):
    B, N, C = xt.shape
    O = wb.shape[0]
    R = _DIST_R
    f = jax.ShapeDtypeStruct
    rspec = lambda t0, t1: pl.BlockSpec((1, t0, t1), lambda b_, r: (b_, r, 0))
    fspec = pl.BlockSpec((1, N, C), lambda b_, r: (b_, 0, 0))
    wspec = lambda s: pl.BlockSpec(s, lambda b_, r: (0,) * len(s))
    return pl.pallas_call(
        _dist_body,
        grid=(B, N // R),
        in_specs=[fspec, rspec(R, C), wspec(wb.shape), wspec(b.shape)],
        out_specs=[rspec(R, N), rspec(R, O)],
        out_shape=[f((B, N, N), jnp.float32), f((B, N, O), jnp.float32)],
    )(xt, xt, wb, b)


def _edge_body(feat_ref, xtr_ref, ybr_ref, wa_ref, x_ref):
    R, K, C = feat_ref.shape[1:]
    O = wa_ref.shape[0]
    g = feat_ref[0] - xtr_ref[0][:, None, :]
    e = _mm(g.reshape(R * K, C), wa_ref[...])
    h = e.reshape(R, K, O) + ybr_ref[0][:, None, :]
    x_ref[0] = jnp.max(_leaky(h), axis=1)


def _edge(feat, xt, yb, wa, R):
    B, N, K, C = feat.shape
    O = wa.shape[0]
    rspec = lambda t0, *t: pl.BlockSpec((1, t0) + t, lambda b_, r: (b_, r) + (0,) * len(t))
    wspec = pl.BlockSpec(wa.shape, lambda b_, r: (0, 0))
    return pl.pallas_call(
        _edge_body,
        grid=(B, N // R),
        in_specs=[rspec(R, K, C), rspec(R, C), rspec(R, O), wspec],
        out_specs=[rspec(R, O)],
        out_shape=[jax.ShapeDtypeStruct((B, N, O), jnp.float32)],
    )(feat, xt, yb, wa)[0]


def _mlp_body(a_ref, b2_ref, wm_ref, bm_ref, h_ref):
    hcat = jnp.concatenate([a_ref[0], b2_ref[0]], axis=1)
    h_ref[0] = jnp.maximum(_mm(hcat, wm_ref[...]) + bm_ref[...][None, :], 0.0)


def _mlp(a, b2, wm, bm):
    B, N, C = a.shape
    O = wm.shape[0]
    bspec = lambda *t: pl.BlockSpec((1,) + t, lambda b_: (b_,) + (0,) * len(t))
    wspec = lambda s: pl.BlockSpec(s, lambda b_: (0,) * len(s))
    return pl.pallas_call(
        _mlp_body,
        grid=(B,),
        in_specs=[bspec(N, C), bspec(N, C), wspec(wm.shape), wspec(bm.shape)],
        out_specs=[bspec(N, O)],
        out_shape=[jax.ShapeDtypeStruct((B, N, O), jnp.float32)],
    )(a, b2, wm, bm)[0]


def _final_body(a_ref, b2_ref, wm_ref, bm_ref, out_ref):
    hcat = jnp.concatenate([a_ref[0], b2_ref[0]], axis=1)
    h = jnp.maximum(_mm(hcat, wm_ref[...]) + bm_ref[...][None, :], 0.0)
    out_ref[0, 0] = jnp.max(h, axis=0)


def _final(a, b2, wm, bm):
    B, N, C = a.shape
    O = wm.shape[0]
    f = jax.ShapeDtypeStruct
    bspec = lambda *t: pl.BlockSpec((1,) + t, lambda b_: (b_,) + (0,) * len(t))
    wspec = lambda s: pl.BlockSpec(s, lambda b_: (0,) * len(s))
    return pl.pallas_call(
        _final_body,
        grid=(B,),
        in_specs=[bspec(N, C), bspec(N, C), wspec(wm.shape), wspec(bm.shape)],
        out_specs=[bspec(1, O)],
        out_shape=[f((B, 1, O), jnp.float32)],
    )(a, b2, wm, bm)[0].reshape(B, O)


def _sc_topk_gather(d_flat, xt_flat):
    P, N = d_flat.shape
    C = xt_flat.shape[1]
    ppw = P // _NW
    nchunks = N // 16
    mesh = plsc.VectorSubcoreMesh(core_axis_name="c", subcore_axis_name="s")

    def body(d_hbm, x_hbm, feat_hbm, row_v, idx_v, rows_v, sem):
        wid = lax.axis_index("s") * _SC_CORES + lax.axis_index("c")

        def point(i, carry):
            p = wid * ppw + i
            base = (p // N) * N
            pltpu.sync_copy(d_hbm.at[p], row_v)

            def chunk(c, kv):
                ak, av = kv
                keys = row_v[pl.ds(c * 16, 16)]

                def merge(_):
                    vals = lax.iota(jnp.int32, 16) + c * 16
                    ck, cv = plsc.sort_key_val(keys, vals, descending=True)
                    m = ak >= ck
                    nk = jnp.where(m, ak, ck)
                    nv = jnp.where(m, av, cv)
                    nk, nv = plsc.sort_key_val(nk, nv)
                    return nk, nv

                return lax.cond(jnp.max(keys) > ak[0], merge,
                                lambda _: (ak, av), 0)

            _, av = lax.fori_loop(
                0, nchunks, chunk,
                (jnp.full((16,), _NEG_HUGE, jnp.float32),
                 jnp.zeros((16,), jnp.int32)))
            idx_v[...] = av + base
            pltpu.async_copy(x_hbm.at[idx_v], rows_v, sem).wait()
            pltpu.sync_copy(rows_v, feat_hbm.at[p])
            return carry

        lax.fori_loop(0, ppw, point, 0)

    return pl.kernel(
        body,
        out_type=jax.ShapeDtypeStruct((P, KNN, C), jnp.float32),
        mesh=mesh,
        compiler_params=pltpu.CompilerParams(needs_layout_passes=False),
        scratch_types=[
            pltpu.VMEM((N,), jnp.float32),
            pltpu.VMEM((KNN,), jnp.int32),
            pltpu.VMEM((KNN, C), jnp.float32),
            pltpu.SemaphoreType.DMA,
        ],
    )(d_flat, xt_flat)


def _edge_stage(xt, wa, wb, b, r_edge):
    B, N, C = xt.shape
    d, yb = _dist_yb(xt, wb, b)
    feat = _sc_topk_gather(d.reshape(B * N, N), xt.reshape(B * N, C))
    return _edge(feat.reshape(B, N, KNN, C), xt, yb, wa, r_edge)


def kernel(x, W1, b1, W2, b2, Wm1, bm1, W3, b3, W4, b4, Wm2, bm2):
    B, C0, N = x.shape
    xt = jnp.swapaxes(x, 1, 2)
    xtp = jnp.pad(xt, ((0, 0), (0, 0), (0, 128 - C0)))
    Wa1 = jnp.pad(W1[:, :C0], ((0, 0), (0, 128 - C0)))
    Wb1 = jnp.pad(W1[:, C0:], ((0, 0), (0, 128 - C0)))

    x1 = _edge_stage(xtp, Wa1, Wb1, b1, 256)
    x2 = _edge_stage(x1, W2[:, :128], W2[:, 128:], b2, 256)
    h = _mlp(x1, x2, Wm1, bm1)
    x3 = _edge_stage(h, W3[:, :1024], W3[:, 1024:], b3, 64)
    x4 = _edge_stage(x3, W4[:, :1024], W4[:, 1024:], b4, 64)
    return _final(x3, x4, Wm2, bm2)


# speedup vs baseline: 1.5415x; 1.5415x over previous
import jax
import jax.numpy as jnp
from jax import lax
from jax.experimental import pallas as pl
from jax.experimental.pallas import tpu as pltpu
from jax.experimental.pallas import tpu_sc as plsc

KNN = 16
_SC_CORES = 2
_SC_SUBCORES = 16
_NW = _SC_CORES * _SC_SUBCORES
_NEG_HUGE = -3.4e38


def _leaky(x):
    return jnp.where(x >= 0, x, 0.2 * x)


def _mm(a, w):
    return lax.dot_general(
        a, w, (((1,), (1,)), ((), ())),
        preferred_element_type=jnp.float32,
        precision=lax.Precision.DEFAULT,
    )


_DIST_R = 256


def _dist_body(xtf_ref, xtr_ref, wb_ref, b_ref, d_ref, yb_ref):
    xtf = xtf_ref[0]
    xtr = xtr_ref[0]
    inner = -2.0 * lax.dot_general(
        xtr, xtf, (((1,), (1,)), ((), ())),
        preferred_element_type=jnp.float32, precision=lax.Precision.DEFAULT)
    xxf = jnp.sum(xtf * xtf, axis=1)
    xxr = jnp.sum(xtr * xtr, axis=1)
    d_ref[0] = -xxr[:, None] - inner - xxf[None, :]
    yb_ref[0] = _mm(xtr, wb_ref[...]) + b_ref[...][None, :]


def _dist_yb(xt, wb, b):
    B, N, C = xt.shape
    O = wb.shape[0]
    R = _DIST_R
    f = jax.ShapeDtypeStruct
    rspec = lambda t0, t1: pl.BlockSpec((1, t0, t1), lambda b_, r: (b_, r, 0))
    fspec = pl.BlockSpec((1, N, C), lambda b_, r: (b_, 0, 0))
    wspec = lambda s: pl.BlockSpec(s, lambda b_, r: (0,) * len(s))
    return pl.pallas_call(
        _dist_body,
        grid=(B, N // R),
        in_specs=[fspec, rspec(R, C), wspec(wb.shape), wspec(b.shape)],
        out_specs=[rspec(R, N), rspec(R, O)],
        out_shape=[f((B, N, N), jnp.float32), f((B, N, O), jnp.float32)],
    )(xt, xt, wb, b)


def _edge_body(feat_ref, xtr_ref, ybr_ref, wa_ref, x_ref):
    R, K, C = feat_ref.shape[1:]
    O = wa_ref.shape[0]
    g = feat_ref[0] - xtr_ref[0][:, None, :]
    e = _mm(g.reshape(R * K, C), wa_ref[...])
    h = e.reshape(R, K, O) + ybr_ref[0][:, None, :]
    x_ref[0] = jnp.max(_leaky(h), axis=1)


def _edge(feat, xt, yb, wa, R):
    B, N, K, C = feat.shape
    O = wa.shape[0]
    rspec = lambda t0, *t: pl.BlockSpec((1, t0) + t, lambda b_, r: (b_, r) + (0,) * len(t))
    wspec = pl.BlockSpec(wa.shape, lambda b_, r: (0, 0))
    return pl.pallas_call(
        _edge_body,
        grid=(B, N // R),
        in_specs=[rspec(R, K, C), rspec(R, C), rspec(R, O), wspec],
        out_specs=[rspec(R, O)],
        out_shape=[jax.ShapeDtypeStruct((B, N, O), jnp.float32)],
    )(feat, xt, yb, wa)[0]


def _mlp_body(a_ref, b2_ref, wm_ref, bm_ref, h_ref):
    hcat = jnp.concatenate([a_ref[0], b2_ref[0]], axis=1)
    h_ref[0] = jnp.maximum(_mm(hcat, wm_ref[...]) + bm_ref[...][None, :], 0.0)


def _mlp(a, b2, wm, bm):
    B, N, C = a.shape
    O = wm.shape[0]
    bspec = lambda *t: pl.BlockSpec((1,) + t, lambda b_: (b_,) + (0,) * len(t))
    wspec = lambda s: pl.BlockSpec(s, lambda b_: (0,) * len(s))
    return pl.pallas_call(
        _mlp_body,
        grid=(B,),
        in_specs=[bspec(N, C), bspec(N, C), wspec(wm.shape), wspec(bm.shape)],
        out_specs=[bspec(N, O)],
        out_shape=[jax.ShapeDtypeStruct((B, N, O), jnp.float32)],
    )(a, b2, wm, bm)[0]


def _final_body(a_ref, b2_ref, wm_ref, bm_ref, out_ref):
    hcat = jnp.concatenate([a_ref[0], b2_ref[0]], axis=1)
    h = jnp.maximum(_mm(hcat, wm_ref[...]) + bm_ref[...][None, :], 0.0)
    out_ref[0, 0] = jnp.max(h, axis=0)


def _final(a, b2, wm, bm):
    B, N, C = a.shape
    O = wm.shape[0]
    f = jax.ShapeDtypeStruct
    bspec = lambda *t: pl.BlockSpec((1,) + t, lambda b_: (b_,) + (0,) * len(t))
    wspec = lambda s: pl.BlockSpec(s, lambda b_: (0,) * len(s))
    return pl.pallas_call(
        _final_body,
        grid=(B,),
        in_specs=[bspec(N, C), bspec(N, C), wspec(wm.shape), wspec(bm.shape)],
        out_specs=[bspec(1, O)],
        out_shape=[f((B, 1, O), jnp.float32)],
    )(a, b2, wm, bm)[0].reshape(B, O)


def _sc_topk_gather(d_flat, xt_flat):
    P, N = d_flat.shape
    C = xt_flat.shape[1]
    ppw = P // _NW
    nchunks = N // 16
    mesh = plsc.VectorSubcoreMesh(core_axis_name="c", subcore_axis_name="s")

    def body(d_hbm, x_hbm, feat_hbm, row0_v, row1_v, idx_v, rows_v,
             sem0, sem1, semg):
        wid = lax.axis_index("s") * _SC_CORES + lax.axis_index("c")
        start = wid * ppw

        def process(p, row_v):
            base = (p // N) * N

            def chunk(c, kv):
                ak, av = kv
                keys = row_v[pl.ds(c * 16, 16)]
                vals = lax.iota(jnp.int32, 16) + c * 16
                ck, cv = plsc.sort_key_val(keys, vals, descending=True)
                m = ak >= ck
                nk = jnp.where(m, ak, ck)
                nv = jnp.where(m, av, cv)
                nk, nv = plsc.sort_key_val(nk, nv)
                return nk, nv

            _, av = lax.fori_loop(
                0, nchunks, chunk,
                (jnp.full((16,), _NEG_HUGE, jnp.float32),
                 jnp.zeros((16,), jnp.int32)))
            idx_v[...] = av + base
            pltpu.async_copy(x_hbm.at[idx_v], rows_v, semg).wait()
            pltpu.sync_copy(rows_v, feat_hbm.at[p])

        npairs = ppw // 2
        pltpu.async_copy(d_hbm.at[start], row0_v, sem0)

        def pair(j, carry):
            p0 = start + 2 * j
            pltpu.async_copy(d_hbm.at[p0 + 1], row1_v, sem1)
            pltpu.make_async_copy(d_hbm.at[p0], row0_v, sem0).wait()
            process(p0, row0_v)

            @pl.when(j + 1 < npairs)
            def _():
                pltpu.async_copy(d_hbm.at[p0 + 2], row0_v, sem0)

            pltpu.make_async_copy(d_hbm.at[p0 + 1], row1_v, sem1).wait()
            process(p0 + 1, row1_v)
            return carry

        lax.fori_loop(0, npairs, pair, 0)

    return pl.kernel(
        body,
        out_type=jax.ShapeDtypeStruct((P, KNN, C), jnp.float32),
        mesh=mesh,
        compiler_params=pltpu.CompilerParams(needs_layout_passes=False),
        scratch_types=[
            pltpu.VMEM((N,), jnp.float32),
            pltpu.VMEM((N,), jnp.float32),
            pltpu.VMEM((KNN,), jnp.int32),
            pltpu.VMEM((KNN, C), jnp.float32),
            pltpu.SemaphoreType.DMA,
            pltpu.SemaphoreType.DMA,
            pltpu.SemaphoreType.DMA,
        ],
    )(d_flat, xt_flat)


def _edge_stage(xt, wa, wb, b, r_edge):
    B, N, C = xt.shape
    d, yb = _dist_yb(xt, wb, b)
    feat = _sc_topk_gather(d.reshape(B * N, N), xt.reshape(B * N, C))
    return _edge(feat.reshape(B, N, KNN, C), xt, yb, wa, r_edge)


def kernel(x, W1, b1, W2, b2, Wm1, bm1, W3, b3, W4, b4, Wm2, bm2):
    B, C0, N = x.shape
    xt = jnp.swapaxes(x, 1, 2)
    xtp = jnp.pad(xt, ((0, 0), (0, 0), (0, 128 - C0)))
    Wa1 = jnp.pad(W1[:, :C0], ((0, 0), (0, 128 - C0)))
    Wb1 = jnp.pad(W1[:, C0:], ((0, 0), (0, 128 - C0)))

    x1 = _edge_stage(xtp, Wa1, Wb1, b1, 256)
    x2 = _edge_stage(x1, W2[:, :128], W2[:, 128:], b2, 256)
    h = _mlp(x1, x2, Wm1, bm1)
    x3 = _edge_stage(h, W3[:, :1024], W3[:, 1024:], b3, 64)
    x4 = _edge_stage(x3, W4[:, :1024], W4[:, 1024:], b4, 64)
    return _final(x3, x4, Wm2, bm2)


# speedup vs baseline: 1.7058x; 1.1066x over previous
import jax
import jax.numpy as jnp
from jax import lax
from jax.experimental import pallas as pl
from jax.experimental.pallas import tpu as pltpu
from jax.experimental.pallas import tpu_sc as plsc

KNN = 16
_SC_CORES = 2
_SC_SUBCORES = 16
_NW = _SC_CORES * _SC_SUBCORES
_NEG_HUGE = -3.4e38


def _leaky(x):
    return jnp.where(x >= 0, x, 0.2 * x)


def _mm(a, w):
    return lax.dot_general(
        a, w, (((1,), (1,)), ((), ())),
        preferred_element_type=jnp.float32,
        precision=lax.Precision.DEFAULT,
    )


_DIST_R = 256


def _dist_body(xtf_ref, xtr_ref, wb_ref, b_ref, d_ref, yb_ref):
    xtf = xtf_ref[0]
    xtr = xtr_ref[0]
    inner = -2.0 * lax.dot_general(
        xtr, xtf, (((1,), (1,)), ((), ())),
        preferred_element_type=jnp.float32, precision=lax.Precision.DEFAULT)
    xxf = jnp.sum(xtf * xtf, axis=1)
    xxr = jnp.sum(xtr * xtr, axis=1)
    d_ref[0] = -xxr[:, None] - inner - xxf[None, :]
    yb_ref[0] = _mm(xtr, wb_ref[...]) + b_ref[...][None, :]


def _dist_yb(xt, wb, b):
    B, N, C = xt.shape
    O = wb.shape[0]
    R = _DIST_R
    f = jax.ShapeDtypeStruct
    rspec = lambda t0, t1: pl.BlockSpec((1, t0, t1), lambda b_, r: (b_, r, 0))
    fspec = pl.BlockSpec((1, N, C), lambda b_, r: (b_, 0, 0))
    wspec = lambda s: pl.BlockSpec(s, lambda b_, r: (0,) * len(s))
    return pl.pallas_call(
        _dist_body,
        grid=(B, N // R),
        in_specs=[fspec, rspec(R, C), wspec(wb.shape), wspec(b.shape)],
        out_specs=[rspec(R, N), rspec(R, O)],
        out_shape=[f((B, N, N), jnp.float32), f((B, N, O), jnp.float32)],
    )(xt, xt, wb, b)


def _edge_body(feat_ref, xtr_ref, ybr_ref, wa_ref, x_ref):
    R, K, C = feat_ref.shape[1:]
    O = wa_ref.shape[0]
    g = feat_ref[0] - xtr_ref[0][:, None, :]
    e = _mm(g.reshape(R * K, C), wa_ref[...])
    h = e.reshape(R, K, O) + ybr_ref[0][:, None, :]
    x_ref[0] = jnp.max(_leaky(h), axis=1)


def _edge(feat, xt, yb, wa, R):
    B, N, K, C = feat.shape
    O = wa.shape[0]
    rspec = lambda t0, *t: pl.BlockSpec((1, t0) + t, lambda b_, r: (b_, r) + (0,) * len(t))
    wspec = pl.BlockSpec(wa.shape, lambda b_, r: (0, 0))
    return pl.pallas_call(
        _edge_body,
        grid=(B, N // R),
        in_specs=[rspec(R, K, C), rspec(R, C), rspec(R, O), wspec],
        out_specs=[rspec(R, O)],
        out_shape=[jax.ShapeDtypeStruct((B, N, O), jnp.float32)],
    )(feat, xt, yb, wa)[0]


def _mlp_body(a_ref, b2_ref, wm_ref, bm_ref, h_ref):
    hcat = jnp.concatenate([a_ref[0], b2_ref[0]], axis=1)
    h_ref[0] = jnp.maximum(_mm(hcat, wm_ref[...]) + bm_ref[...][None, :], 0.0)


def _mlp(a, b2, wm, bm):
    B, N, C = a.shape
    O = wm.shape[0]
    bspec = lambda *t: pl.BlockSpec((1,) + t, lambda b_: (b_,) + (0,) * len(t))
    wspec = lambda s: pl.BlockSpec(s, lambda b_: (0,) * len(s))
    return pl.pallas_call(
        _mlp_body,
        grid=(B,),
        in_specs=[bspec(N, C), bspec(N, C), wspec(wm.shape), wspec(bm.shape)],
        out_specs=[bspec(N, O)],
        out_shape=[jax.ShapeDtypeStruct((B, N, O), jnp.float32)],
    )(a, b2, wm, bm)[0]


def _final_body(a_ref, b2_ref, wm_ref, bm_ref, out_ref):
    hcat = jnp.concatenate([a_ref[0], b2_ref[0]], axis=1)
    h = jnp.maximum(_mm(hcat, wm_ref[...]) + bm_ref[...][None, :], 0.0)
    out_ref[0, 0] = jnp.max(h, axis=0)


def _final(a, b2, wm, bm):
    B, N, C = a.shape
    O = wm.shape[0]
    f = jax.ShapeDtypeStruct
    bspec = lambda *t: pl.BlockSpec((1,) + t, lambda b_: (b_,) + (0,) * len(t))
    wspec = lambda s: pl.BlockSpec(s, lambda b_: (0,) * len(s))
    return pl.pallas_call(
        _final_body,
        grid=(B,),
        in_specs=[bspec(N, C), bspec(N, C), wspec(wm.shape), wspec(bm.shape)],
        out_specs=[bspec(1, O)],
        out_shape=[f((B, 1, O), jnp.float32)],
    )(a, b2, wm, bm)[0].reshape(B, O)


def _sc_topk_gather(d_flat, xt_flat):
    P, N = d_flat.shape
    C = xt_flat.shape[1]
    ppw = P // _NW
    nchunks = N // 16
    mesh = plsc.VectorSubcoreMesh(core_axis_name="c", subcore_axis_name="s")

    def body(d_hbm, x_hbm, feat_hbm, row0_v, row1_v, idx0_v, idx1_v,
             rows0_v, rows1_v, sem0, sem1, semg0, semg1, semw0, semw1):
        wid = lax.axis_index("s") * _SC_CORES + lax.axis_index("c")
        start = wid * ppw

        def process(p, row_v, idx_v, rows_v, semg, semw, has_prev):
            base = (p // N) * N

            def chunk(c, kv):
                ak, av = kv
                keys = row_v[pl.ds(c * 16, 16)]
                vals = lax.iota(jnp.int32, 16) + c * 16
                ck, cv = plsc.sort_key_val(keys, vals, descending=True)
                m = ak >= ck
                nk = jnp.where(m, ak, ck)
                nv = jnp.where(m, av, cv)
                nk, nv = plsc.sort_key_val(nk, nv)
                return nk, nv

            _, av = lax.fori_loop(
                0, nchunks, chunk,
                (jnp.full((16,), _NEG_HUGE, jnp.float32),
                 jnp.zeros((16,), jnp.int32)))
            idx_v[...] = av + base

            @pl.when(has_prev)
            def _():
                pltpu.make_async_copy(rows_v, feat_hbm.at[p - 2], semw).wait()

            pltpu.async_copy(x_hbm.at[idx_v], rows_v, semg).wait()
            pltpu.async_copy(rows_v, feat_hbm.at[p], semw)

        npairs = ppw // 2
        pltpu.async_copy(d_hbm.at[start], row0_v, sem0)

        def pair(j, carry):
            p0 = start + 2 * j
            pltpu.async_copy(d_hbm.at[p0 + 1], row1_v, sem1)
            pltpu.make_async_copy(d_hbm.at[p0], row0_v, sem0).wait()
            process(p0, row0_v, idx0_v, rows0_v, semg0, semw0, j > 0)

            @pl.when(j + 1 < npairs)
            def _():
                pltpu.async_copy(d_hbm.at[p0 + 2], row0_v, sem0)

            pltpu.make_async_copy(d_hbm.at[p0 + 1], row1_v, sem1).wait()
            process(p0 + 1, row1_v, idx1_v, rows1_v, semg1, semw1, j > 0)
            return carry

        lax.fori_loop(0, npairs, pair, 0)
        pltpu.make_async_copy(
            rows0_v, feat_hbm.at[start + ppw - 2], semw0).wait()
        pltpu.make_async_copy(
            rows1_v, feat_hbm.at[start + ppw - 1], semw1).wait()

    return pl.kernel(
        body,
        out_type=jax.ShapeDtypeStruct((P, KNN, C), jnp.float32),
        mesh=mesh,
        compiler_params=pltpu.CompilerParams(needs_layout_passes=False),
        scratch_types=[
            pltpu.VMEM((N,), jnp.float32),
            pltpu.VMEM((N,), jnp.float32),
            pltpu.VMEM((KNN,), jnp.int32),
            pltpu.VMEM((KNN,), jnp.int32),
            pltpu.VMEM((KNN, C), jnp.float32),
            pltpu.VMEM((KNN, C), jnp.float32),
            pltpu.SemaphoreType.DMA,
            pltpu.SemaphoreType.DMA,
            pltpu.SemaphoreType.DMA,
            pltpu.SemaphoreType.DMA,
            pltpu.SemaphoreType.DMA,
            pltpu.SemaphoreType.DMA,
        ],
    )(d_flat, xt_flat)


def _edge_stage(xt, wa, wb, b, r_edge):
    B, N, C = xt.shape
    d, yb = _dist_yb(xt, wb, b)
    feat = _sc_topk_gather(d.reshape(B * N, N), xt.reshape(B * N, C))
    return _edge(feat.reshape(B, N, KNN, C), xt, yb, wa, r_edge)


def kernel(x, W1, b1, W2, b2, Wm1, bm1, W3, b3, W4, b4, Wm2, bm2):
    B, C0, N = x.shape
    xt = jnp.swapaxes(x, 1, 2)
    xtp = jnp.pad(xt, ((0, 0), (0, 0), (0, 128 - C0)))
    Wa1 = jnp.pad(W1[:, :C0], ((0, 0), (0, 128 - C0)))
    Wb1 = jnp.pad(W1[:, C0:], ((0, 0), (0, 128 - C0)))

    x1 = _edge_stage(xtp, Wa1, Wb1, b1, 256)
    x2 = _edge_stage(x1, W2[:, :128], W2[:, 128:], b2, 256)
    h = _mlp(x1, x2, Wm1, bm1)
    x3 = _edge_stage(h, W3[:, :1024], W3[:, 1024:], b3, 64)
    x4 = _edge_stage(x3, W4[:, :1024], W4[:, 1024:], b4, 64)
    return _final(x3, x4, Wm2, bm2)
